# baseline (device time: 429434 ns/iter reference)
import jax
import jax.numpy as jnp
from jax import lax
from jax.experimental import pallas as pl
from jax.experimental.pallas import tpu as pltpu

T = 2048
D = 1024
E_LOCAL = 4
F = 2048

TB = 512
FB = 1024
N_TB = 2 * T // TB
N_FB = F // FB


def _exchange(x, assign2d):

    def body(x_ref, a_ref, xg_ref, ag_ref, send_sems, recv_sems):
        my_x = lax.axis_index("x")
        my_y = lax.axis_index("y")
        peer = (my_x, 1 - my_y)

        barrier = pltpu.get_barrier_semaphore()
        pl.semaphore_signal(barrier, inc=1, device_id=peer,
                            device_id_type=pl.DeviceIdType.MESH)
        pl.semaphore_wait(barrier, 1)

        my_off = my_y * T
        xg_ref[pl.ds(my_off, T), :] = x_ref[:, :]
        ag_ref[pl.ds(my_off, T), :] = a_ref[:, :]

        rdma_x = pltpu.make_async_remote_copy(
            src_ref=x_ref,
            dst_ref=xg_ref.at[pl.ds(my_off, T), :],
            send_sem=send_sems.at[0],
            recv_sem=recv_sems.at[0],
            device_id=peer,
            device_id_type=pl.DeviceIdType.MESH,
        )
        rdma_a = pltpu.make_async_remote_copy(
            src_ref=a_ref,
            dst_ref=ag_ref.at[pl.ds(my_off, T), :],
            send_sem=send_sems.at[1],
            recv_sem=recv_sems.at[1],
            device_id=peer,
            device_id_type=pl.DeviceIdType.MESH,
        )
        rdma_x.start()
        rdma_a.start()
        rdma_x.wait()
        rdma_a.wait()

    return pl.pallas_call(
        body,
        out_shape=[
            jax.ShapeDtypeStruct((2 * T, D), jnp.float32),
            jax.ShapeDtypeStruct((2 * T, 1), jnp.int32),
        ],
        in_specs=[
            pl.BlockSpec(memory_space=pltpu.VMEM),
            pl.BlockSpec(memory_space=pltpu.VMEM),
        ],
        out_specs=[
            pl.BlockSpec(memory_space=pltpu.VMEM),
            pl.BlockSpec(memory_space=pltpu.VMEM),
        ],
        scratch_shapes=[
            pltpu.SemaphoreType.DMA((2,)),
            pltpu.SemaphoreType.DMA((2,)),
        ],
        compiler_params=pltpu.CompilerParams(collective_id=0),
    )(x, assign2d)


def _moe(xg, ag, W1, W2):

    def body(xg_ref, ag_ref, w1_ref, w2_ref, out_ref):
        e = pl.program_id(1)
        fb = pl.program_id(2)
        my_y = lax.axis_index("y")

        h = jnp.maximum(
            jnp.dot(xg_ref[:, :], w1_ref[0],
                    preferred_element_type=jnp.float32),
            0.0,
        )
        o = jnp.dot(h, w2_ref[0], preferred_element_type=jnp.float32)
        mask = (ag_ref[:, :] == my_y * E_LOCAL + e).astype(jnp.float32)
        contrib = mask * o

        first = (e == 0) & (fb == 0)

        @pl.when(first)
        def _():
            out_ref[:, :] = contrib

        @pl.when(jnp.logical_not(first))
        def _():
            out_ref[:, :] += contrib

    grid = (N_TB, E_LOCAL, N_FB)
    return pl.pallas_call(
        body,
        grid=grid,
        in_specs=[
            pl.BlockSpec((TB, D), lambda tb, e, fb: (tb, 0)),
            pl.BlockSpec((TB, 1), lambda tb, e, fb: (tb, 0)),
            pl.BlockSpec((1, D, FB), lambda tb, e, fb: (e, 0, fb)),
            pl.BlockSpec((1, FB, D), lambda tb, e, fb: (e, fb, 0)),
        ],
        out_specs=pl.BlockSpec((TB, D), lambda tb, e, fb: (tb, 0)),
        out_shape=jax.ShapeDtypeStruct((2 * T, D), jnp.float32),
    )(xg, ag, W1, W2)


def _combine(partial):

    def body(p_ref, out_ref, comm_ref, send_sem, recv_sem):
        my_x = lax.axis_index("x")
        my_y = lax.axis_index("y")
        peer = (my_x, 1 - my_y)

        barrier = pltpu.get_barrier_semaphore()
        pl.semaphore_signal(barrier, inc=1, device_id=peer,
                            device_id_type=pl.DeviceIdType.MESH)
        pl.semaphore_wait(barrier, 1)

        peer_off = (1 - my_y) * T
        rdma = pltpu.make_async_remote_copy(
            src_ref=p_ref.at[pl.ds(peer_off, T), :],
            dst_ref=comm_ref,
            send_sem=send_sem,
            recv_sem=recv_sem,
            device_id=peer,
            device_id_type=pl.DeviceIdType.MESH,
        )
        rdma.start()
        rdma.wait()

        out_ref[:, :] = p_ref[pl.ds(my_y * T, T), :] + comm_ref[:, :]

    return pl.pallas_call(
        body,
        out_shape=jax.ShapeDtypeStruct((T, D), jnp.float32),
        in_specs=[pl.BlockSpec(memory_space=pltpu.VMEM)],
        out_specs=pl.BlockSpec(memory_space=pltpu.VMEM),
        scratch_shapes=[
            pltpu.VMEM((T, D), jnp.float32),
            pltpu.SemaphoreType.DMA,
            pltpu.SemaphoreType.DMA,
        ],
        compiler_params=pltpu.CompilerParams(collective_id=1),
    )(partial)


def kernel(x, assign, W1, W2):
    assign2d = assign.reshape(T, 1)
    xg, ag = _exchange(x, assign2d)
    partial = _moe(xg, ag, W1, W2)
    return _combine(partial)


# device time: 380848 ns/iter; 1.1276x vs baseline; 1.1276x over previous
import jax
import jax.numpy as jnp
from jax import lax
from jax.experimental import pallas as pl
from jax.experimental.pallas import tpu as pltpu

T = 2048
D = 1024
E_LOCAL = 4
F = 2048

TB = 512
FB = 1024
N_TB = 2 * T // TB
N_FB = F // FB


def _exchange(x, assign2d):

    def body(x_ref, a_ref, xg_ref, ag_ref, send_sems, recv_sems):
        my_x = lax.axis_index("x")
        my_y = lax.axis_index("y")
        peer = (my_x, 1 - my_y)

        barrier = pltpu.get_barrier_semaphore()
        pl.semaphore_signal(barrier, inc=1, device_id=peer,
                            device_id_type=pl.DeviceIdType.MESH)
        pl.semaphore_wait(barrier, 1)

        my_off = my_y * T
        xg_ref[pl.ds(my_off, T), :] = x_ref[:, :]
        ag_ref[pl.ds(my_off, T), :] = a_ref[:, :]

        rdma_x = pltpu.make_async_remote_copy(
            src_ref=x_ref,
            dst_ref=xg_ref.at[pl.ds(my_off, T), :],
            send_sem=send_sems.at[0],
            recv_sem=recv_sems.at[0],
            device_id=peer,
            device_id_type=pl.DeviceIdType.MESH,
        )
        rdma_a = pltpu.make_async_remote_copy(
            src_ref=a_ref,
            dst_ref=ag_ref.at[pl.ds(my_off, T), :],
            send_sem=send_sems.at[1],
            recv_sem=recv_sems.at[1],
            device_id=peer,
            device_id_type=pl.DeviceIdType.MESH,
        )
        rdma_x.start()
        rdma_a.start()
        rdma_x.wait()
        rdma_a.wait()

    return pl.pallas_call(
        body,
        out_shape=[
            jax.ShapeDtypeStruct((2 * T, D), x.dtype),
            jax.ShapeDtypeStruct((2 * T, 1), jnp.int32),
        ],
        in_specs=[
            pl.BlockSpec(memory_space=pltpu.VMEM),
            pl.BlockSpec(memory_space=pltpu.VMEM),
        ],
        out_specs=[
            pl.BlockSpec(memory_space=pltpu.VMEM),
            pl.BlockSpec(memory_space=pltpu.VMEM),
        ],
        scratch_shapes=[
            pltpu.SemaphoreType.DMA((2,)),
            pltpu.SemaphoreType.DMA((2,)),
        ],
        compiler_params=pltpu.CompilerParams(collective_id=0),
    )(x, assign2d)


def _moe(xg, ag, W1, W2):

    def body(xg_ref, ag_ref, w1_ref, w2_ref, out_ref):
        e = pl.program_id(0)
        fb = pl.program_id(1)
        tb = pl.program_id(2)
        my_y = lax.axis_index("y")

        h = jnp.maximum(
            jnp.dot(xg_ref[:, :], w1_ref[0],
                    preferred_element_type=jnp.float32),
            0.0,
        )
        o = jnp.dot(h.astype(jnp.bfloat16), w2_ref[0],
                    preferred_element_type=jnp.float32)
        mask = (ag_ref[:, :] == my_y * E_LOCAL + e).astype(jnp.float32)
        contrib = mask * o

        first = (e == 0) & (fb == 0)
        off = tb * TB

        @pl.when(first)
        def _():
            out_ref[pl.ds(off, TB), :] = contrib

        @pl.when(jnp.logical_not(first))
        def _():
            out_ref[pl.ds(off, TB), :] += contrib

    grid = (E_LOCAL, N_FB, N_TB)
    return pl.pallas_call(
        body,
        grid=grid,
        in_specs=[
            pl.BlockSpec((TB, D), lambda e, fb, tb: (tb, 0)),
            pl.BlockSpec((TB, 1), lambda e, fb, tb: (tb, 0)),
            pl.BlockSpec((1, D, FB), lambda e, fb, tb: (e, 0, fb)),
            pl.BlockSpec((1, FB, D), lambda e, fb, tb: (e, fb, 0)),
        ],
        out_specs=pl.BlockSpec(memory_space=pltpu.VMEM),
        out_shape=jax.ShapeDtypeStruct((2 * T, D), jnp.float32),
    )(xg, ag, W1, W2)


def _combine(partial):

    def body(p_ref, out_ref, comm_ref, send_sem, recv_sem):
        my_x = lax.axis_index("x")
        my_y = lax.axis_index("y")
        peer = (my_x, 1 - my_y)

        barrier = pltpu.get_barrier_semaphore()
        pl.semaphore_signal(barrier, inc=1, device_id=peer,
                            device_id_type=pl.DeviceIdType.MESH)
        pl.semaphore_wait(barrier, 1)

        peer_off = (1 - my_y) * T
        rdma = pltpu.make_async_remote_copy(
            src_ref=p_ref.at[pl.ds(peer_off, T), :],
            dst_ref=comm_ref,
            send_sem=send_sem,
            recv_sem=recv_sem,
            device_id=peer,
            device_id_type=pl.DeviceIdType.MESH,
        )
        rdma.start()
        rdma.wait()

        out_ref[:, :] = p_ref[pl.ds(my_y * T, T), :] + comm_ref[:, :]

    return pl.pallas_call(
        body,
        out_shape=jax.ShapeDtypeStruct((T, D), jnp.float32),
        in_specs=[pl.BlockSpec(memory_space=pltpu.VMEM)],
        out_specs=pl.BlockSpec(memory_space=pltpu.VMEM),
        scratch_shapes=[
            pltpu.VMEM((T, D), jnp.float32),
            pltpu.SemaphoreType.DMA,
            pltpu.SemaphoreType.DMA,
        ],
        compiler_params=pltpu.CompilerParams(collective_id=1),
    )(partial)


def kernel(x, assign, W1, W2):
    assign2d = assign.reshape(T, 1)
    xb = x.astype(jnp.bfloat16)
    w1b = W1.astype(jnp.bfloat16)
    w2b = W2.astype(jnp.bfloat16)
    xg, ag = _exchange(xb, assign2d)
    partial = _moe(xg, ag, w1b, w2b)
    return _combine(partial)


# device time: 262840 ns/iter; 1.6338x vs baseline; 1.4490x over previous
import jax
import jax.numpy as jnp
from jax import lax
from jax.experimental import pallas as pl
from jax.experimental.pallas import tpu as pltpu

T = 2048
D = 1024
E_LOCAL = 4
F = 2048

TBK = 1024
N_TB = T // TBK
FB = 1024
N_FB = F // FB

_NW = 16


def _exchange_cast(x16, assign2d, W1, W2):

    def body(x_ref, a_ref, w1_ref, w2_ref,
             xg_ref, ag_ref, w1b_ref, w2b_ref, send_sems, recv_sems):
        i = pl.program_id(0)
        my_x = lax.axis_index("x")
        my_y = lax.axis_index("y")
        peer = (my_x, 1 - my_y)
        my_off = my_y * T

        rx = pltpu.make_async_remote_copy(
            src_ref=x_ref,
            dst_ref=xg_ref.at[pl.ds(my_off, T), :],
            send_sem=send_sems.at[0],
            recv_sem=recv_sems.at[0],
            device_id=peer,
            device_id_type=pl.DeviceIdType.MESH,
        )
        ra = pltpu.make_async_remote_copy(
            src_ref=a_ref,
            dst_ref=ag_ref.at[pl.ds(my_off, T), :],
            send_sem=send_sems.at[1],
            recv_sem=recv_sems.at[1],
            device_id=peer,
            device_id_type=pl.DeviceIdType.MESH,
        )

        @pl.when(i == 0)
        def _():
            barrier = pltpu.get_barrier_semaphore()
            pl.semaphore_signal(barrier, inc=1, device_id=peer,
                                device_id_type=pl.DeviceIdType.MESH)
            pl.semaphore_wait(barrier, 1)
            xg_ref[pl.ds(my_off, T), :] = x_ref[:, :]
            ag_ref[pl.ds(my_off, T), :] = a_ref[:, :]
            rx.start()
            ra.start()

        @pl.when(i < 8)
        def _():
            w1b_ref[0] = w1_ref[0].astype(jnp.bfloat16)

        @pl.when(i >= 8)
        def _():
            w2b_ref[0] = w2_ref[0].astype(jnp.bfloat16)

        @pl.when(i == _NW - 1)
        def _():
            rx.wait()
            ra.wait()

    i1 = lambda i: jnp.minimum(i, 7)
    i2 = lambda i: jnp.maximum(i - 8, 0)
    return pl.pallas_call(
        body,
        grid=(_NW,),
        in_specs=[
            pl.BlockSpec(memory_space=pltpu.VMEM),
            pl.BlockSpec(memory_space=pltpu.VMEM),
            pl.BlockSpec((1, D, FB), lambda i: (i1(i) // 2, 0, i1(i) % 2)),
            pl.BlockSpec((1, FB, D), lambda i: (i2(i) // 2, i2(i) % 2, 0)),
        ],
        out_specs=[
            pl.BlockSpec(memory_space=pltpu.VMEM),
            pl.BlockSpec(memory_space=pltpu.VMEM),
            pl.BlockSpec((1, D, FB), lambda i: (i1(i) // 2, 0, i1(i) % 2)),
            pl.BlockSpec((1, FB, D), lambda i: (i2(i) // 2, i2(i) % 2, 0)),
        ],
        out_shape=[
            jax.ShapeDtypeStruct((2 * T, D), jnp.bfloat16),
            jax.ShapeDtypeStruct((2 * T, 1), jnp.int32),
            jax.ShapeDtypeStruct((E_LOCAL, D, F), jnp.bfloat16),
            jax.ShapeDtypeStruct((E_LOCAL, F, D), jnp.bfloat16),
        ],
        scratch_shapes=[
            pltpu.SemaphoreType.DMA((2,)),
            pltpu.SemaphoreType.DMA((2,)),
        ],
        compiler_params=pltpu.CompilerParams(collective_id=0),
    )(x16, assign2d, W1, W2)


def _moe_combine(xg, ag, w1b, w2b):

    def body(xg_ref, ag_ref, w1_ref, w2_ref, out_ref,
             pbuf, rbuf, send_sems, recv_sems):
        phase = pl.program_id(0)
        tbp = pl.program_id(1)
        e = pl.program_id(2)
        fb = pl.program_id(3)
        my_x = lax.axis_index("x")
        my_y = lax.axis_index("y")
        peer = (my_x, 1 - my_y)

        @pl.when((phase == 0) & (tbp == 0) & (e == 0) & (fb == 0))
        def _():
            barrier = pltpu.get_barrier_semaphore()
            pl.semaphore_signal(barrier, inc=1, device_id=peer,
                                device_id_type=pl.DeviceIdType.MESH)
            pl.semaphore_wait(barrier, 1)

        my_off = my_y * T
        peer_off = (1 - my_y) * T
        cur_off = jnp.where(phase == 0, my_off, peer_off) + tbp * TBK

        xb = xg_ref[pl.ds(cur_off, TBK), :]
        h = jnp.maximum(
            jnp.dot(xb, w1_ref[0], preferred_element_type=jnp.float32), 0.0)
        o = jnp.dot(h.astype(jnp.bfloat16), w2_ref[0],
                    preferred_element_type=jnp.float32)
        mask = (ag_ref[pl.ds(cur_off, TBK), :] == my_y * E_LOCAL + e)
        contrib = mask.astype(jnp.float32) * o

        init = (e == 0) & (fb == 0)
        out_off = tbp * TBK

        @pl.when((phase == 0) & init)
        def _():
            out_ref[pl.ds(out_off, TBK), :] = contrib

        @pl.when((phase == 0) & jnp.logical_not(init))
        def _():
            out_ref[pl.ds(out_off, TBK), :] += contrib

        @pl.when((phase == 1) & init)
        def _():
            pbuf[pl.ds(out_off, TBK), :] = contrib.astype(jnp.bfloat16)

        @pl.when((phase == 1) & jnp.logical_not(init))
        def _():
            pbuf[pl.ds(out_off, TBK), :] += contrib.astype(jnp.bfloat16)

        last_ef = (e == E_LOCAL - 1) & (fb == N_FB - 1)
        for slot in range(N_TB):
            rdma = pltpu.make_async_remote_copy(
                src_ref=pbuf.at[pl.ds(slot * TBK, TBK), :],
                dst_ref=rbuf.at[pl.ds(slot * TBK, TBK), :],
                send_sem=send_sems.at[slot],
                recv_sem=recv_sems.at[slot],
                device_id=peer,
                device_id_type=pl.DeviceIdType.MESH,
            )

            @pl.when((phase == 1) & (tbp == slot) & last_ef)
            def _():
                rdma.start()

            @pl.when((phase == 1) & (tbp == N_TB - 1) & last_ef)
            def _():
                rdma.wait_send()
                rdma.wait_recv()

        @pl.when((phase == 1) & (tbp == N_TB - 1) & last_ef)
        def _():
            out_ref[:, :] += rbuf[:, :].astype(jnp.float32)

    grid = (2, N_TB, E_LOCAL, N_FB)
    return pl.pallas_call(
        body,
        grid=grid,
        in_specs=[
            pl.BlockSpec(memory_space=pltpu.VMEM),
            pl.BlockSpec(memory_space=pltpu.VMEM),
            pl.BlockSpec((1, D, FB), lambda p, tb, e, fb: (e, 0, fb)),
            pl.BlockSpec((1, FB, D), lambda p, tb, e, fb: (e, fb, 0)),
        ],
        out_specs=pl.BlockSpec(memory_space=pltpu.VMEM),
        out_shape=jax.ShapeDtypeStruct((T, D), jnp.float32),
        scratch_shapes=[
            pltpu.VMEM((T, D), jnp.bfloat16),
            pltpu.VMEM((T, D), jnp.bfloat16),
            pltpu.SemaphoreType.DMA((N_TB,)),
            pltpu.SemaphoreType.DMA((N_TB,)),
        ],
        compiler_params=pltpu.CompilerParams(collective_id=1),
    )(xg, ag, w1b, w2b)


def kernel(x, assign, W1, W2):
    x16 = x.astype(jnp.bfloat16)
    assign2d = assign.reshape(T, 1)
    xg, ag, w1b, w2b = _exchange_cast(x16, assign2d, W1, W2)
    return _moe_combine(xg, ag, w1b, w2b)


# device time: 219416 ns/iter; 1.9572x vs baseline; 1.1979x over previous
import jax
import jax.numpy as jnp
from jax import lax
from jax.experimental import pallas as pl
from jax.experimental.pallas import tpu as pltpu

T = 2048
D = 1024
E_LOCAL = 4
F = 2048

TBK = 1024
N_TB = T // TBK
FB = 512
N_FB = F // FB


def _fused(x16, assign2d, W1, W2):
    def body(x_ref, a_ref, w1_ref, w2_ref, out_ref,
             xg, ag, pbuf, rbuf, xa_send, xa_recv, p_send, p_recv):
        phase = pl.program_id(0)
        tbp = pl.program_id(1)
        e = pl.program_id(2)
        fb = pl.program_id(3)
        my_x = lax.axis_index("x")
        my_y = lax.axis_index("y")
        peer = (my_x, 1 - my_y)
        my_off = my_y * T
        peer_off = (1 - my_y) * T

        rx = pltpu.make_async_remote_copy(
            src_ref=x_ref,
            dst_ref=xg.at[pl.ds(my_off, T), :],
            send_sem=xa_send.at[0],
            recv_sem=xa_recv.at[0],
            device_id=peer,
            device_id_type=pl.DeviceIdType.MESH,
        )
        ra = pltpu.make_async_remote_copy(
            src_ref=a_ref,
            dst_ref=ag.at[pl.ds(my_off, T), :],
            send_sem=xa_send.at[1],
            recv_sem=xa_recv.at[1],
            device_id=peer,
            device_id_type=pl.DeviceIdType.MESH,
        )

        @pl.when((phase == 0) & (tbp == 0) & (e == 0) & (fb == 0))
        def _():
            barrier = pltpu.get_barrier_semaphore()
            pl.semaphore_signal(barrier, inc=1, device_id=peer,
                                device_id_type=pl.DeviceIdType.MESH)
            pl.semaphore_wait(barrier, 1)
            xg[pl.ds(my_off, T), :] = x_ref[:, :]
            ag[pl.ds(my_off, T), :] = a_ref[:, :]
            rx.start()
            ra.start()

        @pl.when((phase == 1) & (tbp == 0) & (e == 0) & (fb == 0))
        def _():
            rx.wait_recv()
            ra.wait_recv()

        cur_off = jnp.where(phase == 0, my_off, peer_off) + tbp * TBK
        xb = xg[pl.ds(cur_off, TBK), :]
        h = jnp.maximum(
            jnp.dot(xb, w1_ref[0], preferred_element_type=jnp.float32), 0.0)
        o = jnp.dot(h, w2_ref[0], preferred_element_type=jnp.float32)
        mask = (ag[pl.ds(cur_off, TBK), :] == my_y * E_LOCAL + e)
        contrib = mask.astype(jnp.float32) * o

        init = (e == 0) & (fb == 0)
        out_off = tbp * TBK

        @pl.when((phase == 0) & init)
        def _():
            out_ref[pl.ds(out_off, TBK), :] = contrib

        @pl.when((phase == 0) & jnp.logical_not(init))
        def _():
            out_ref[pl.ds(out_off, TBK), :] += contrib

        @pl.when((phase == 1) & init)
        def _():
            pbuf[pl.ds(out_off, TBK), :] = contrib.astype(jnp.bfloat16)

        @pl.when((phase == 1) & jnp.logical_not(init))
        def _():
            pbuf[pl.ds(out_off, TBK), :] += contrib.astype(jnp.bfloat16)

        last_ef = (e == E_LOCAL - 1) & (fb == N_FB - 1)
        last_step = (phase == 1) & (tbp == N_TB - 1) & last_ef
        for slot in range(N_TB):
            rdma = pltpu.make_async_remote_copy(
                src_ref=pbuf.at[pl.ds(slot * TBK, TBK), :],
                dst_ref=rbuf.at[pl.ds(slot * TBK, TBK), :],
                send_sem=p_send.at[slot],
                recv_sem=p_recv.at[slot],
                device_id=peer,
                device_id_type=pl.DeviceIdType.MESH,
            )

            @pl.when((phase == 1) & (tbp == slot) & last_ef)
            def _():
                rdma.start()

            @pl.when(last_step)
            def _():
                rdma.wait_send()
                rdma.wait_recv()

        @pl.when(last_step)
        def _():
            rx.wait_send()
            ra.wait_send()
            out_ref[:, :] += rbuf[:, :].astype(jnp.float32)

    grid = (2, N_TB, E_LOCAL, N_FB)
    return pl.pallas_call(
        body,
        grid=grid,
        in_specs=[
            pl.BlockSpec(memory_space=pltpu.VMEM),
            pl.BlockSpec(memory_space=pltpu.VMEM),
            pl.BlockSpec((1, D, FB), lambda p, tb, e, fb: (e, 0, fb)),
            pl.BlockSpec((1, FB, D), lambda p, tb, e, fb: (e, fb, 0)),
        ],
        out_specs=pl.BlockSpec(memory_space=pltpu.VMEM),
        out_shape=jax.ShapeDtypeStruct((T, D), jnp.float32),
        scratch_shapes=[
            pltpu.VMEM((2 * T, D), jnp.bfloat16),
            pltpu.VMEM((2 * T, 1), jnp.int32),
            pltpu.VMEM((T, D), jnp.bfloat16),
            pltpu.VMEM((T, D), jnp.bfloat16),
            pltpu.SemaphoreType.DMA((2,)),
            pltpu.SemaphoreType.DMA((2,)),
            pltpu.SemaphoreType.DMA((N_TB,)),
            pltpu.SemaphoreType.DMA((N_TB,)),
        ],
        compiler_params=pltpu.CompilerParams(collective_id=0),
    )(x16, assign2d, W1, W2)


def kernel(x, assign, W1, W2):
    x16 = x.astype(jnp.bfloat16)
    assign2d = assign.reshape(T, 1)
    return _fused(x16, assign2d, W1, W2)


# device time: 161535 ns/iter; 2.6585x vs baseline; 1.3583x over previous
import jax
import jax.numpy as jnp
from jax import lax
from jax.experimental import pallas as pl
from jax.experimental.pallas import tpu as pltpu

T = 2048
D = 1024
E_LOCAL = 4
F = 2048

Q = 1024
SB = 512
N_SB = Q // SB
FB = 512
N_FB = F // FB


def _fused(x16, assign2d, W1, W2):
    def body(x_ref, a_ref, w1_ref, w2_ref, out_ref,
             xp, ap, pbufA, pbufB, rbufA, rbufBy, rbufBd,
             xa_send, xa_recv, psA, prA, psBy, prBy, psBd, prBd):
        phase = pl.program_id(0)
        sub = pl.program_id(1)
        e = pl.program_id(2)
        fb = pl.program_id(3)
        my_x = lax.axis_index("x")
        my_y = lax.axis_index("y")
        xpeer = (1 - my_x, my_y)
        ypeer = (my_x, 1 - my_y)
        diag = (1 - my_x, 1 - my_y)

        q_off = my_x * Q

        rx = pltpu.make_async_remote_copy(
            src_ref=x_ref.at[pl.ds(q_off, Q), :],
            dst_ref=xp,
            send_sem=xa_send.at[0],
            recv_sem=xa_recv.at[0],
            device_id=ypeer,
            device_id_type=pl.DeviceIdType.MESH,
        )
        ra = pltpu.make_async_remote_copy(
            src_ref=a_ref.at[pl.ds(q_off, Q), :],
            dst_ref=ap,
            send_sem=xa_send.at[1],
            recv_sem=xa_recv.at[1],
            device_id=ypeer,
            device_id_type=pl.DeviceIdType.MESH,
        )

        @pl.when((phase == 0) & (sub == 0) & (e == 0) & (fb == 0))
        def _():
            barrier = pltpu.get_barrier_semaphore()
            for nbr in (xpeer, ypeer, diag):
                pl.semaphore_signal(barrier, inc=1, device_id=nbr,
                                    device_id_type=pl.DeviceIdType.MESH)
            pl.semaphore_wait(barrier, 3)
            rx.start()
            ra.start()

        @pl.when((phase == 1) & (sub == 0) & (e == 0) & (fb == 0))
        def _():
            rx.wait_recv()
            ra.wait_recv()

        row = q_off + sub * SB

        def tile(src_x, src_a, dst):
            xs = src_x
            h = jnp.maximum(
                jnp.dot(xs, w1_ref[0], preferred_element_type=jnp.float32),
                0.0)
            o = jnp.dot(h, w2_ref[0], preferred_element_type=jnp.float32)
            mask = (src_a == my_y * E_LOCAL + e)
            contrib = (mask.astype(jnp.float32) * o).astype(jnp.bfloat16)
            init = (e == 0) & (fb == 0)

            @pl.when(init)
            def _():
                dst[pl.ds(sub * SB, SB), :] = contrib

            @pl.when(jnp.logical_not(init))
            def _():
                dst[pl.ds(sub * SB, SB), :] += contrib

        @pl.when(phase == 0)
        def _():
            tile(x_ref[pl.ds(row, SB), :], a_ref[pl.ds(row, SB), :], pbufA)

        @pl.when(phase == 1)
        def _():
            tile(xp[pl.ds(sub * SB, SB), :], ap[pl.ds(sub * SB, SB), :], pbufB)

        sweep_done = (e == E_LOCAL - 1) & (fb == N_FB - 1)
        last_step = (phase == 1) & (sub == N_SB - 1) & sweep_done

        for slot in range(N_SB):
            sl = pl.ds(slot * SB, SB)
            send_A = pltpu.make_async_remote_copy(
                src_ref=pbufA.at[sl, :], dst_ref=rbufA.at[sl, :],
                send_sem=psA.at[slot], recv_sem=prA.at[slot],
                device_id=xpeer, device_id_type=pl.DeviceIdType.MESH,
            )
            send_By = pltpu.make_async_remote_copy(
                src_ref=pbufB.at[sl, :], dst_ref=rbufBy.at[sl, :],
                send_sem=psBy.at[slot], recv_sem=prBy.at[slot],
                device_id=ypeer, device_id_type=pl.DeviceIdType.MESH,
            )
            send_Bd = pltpu.make_async_remote_copy(
                src_ref=pbufB.at[sl, :], dst_ref=rbufBd.at[sl, :],
                send_sem=psBd.at[slot], recv_sem=prBd.at[slot],
                device_id=diag, device_id_type=pl.DeviceIdType.MESH,
            )

            @pl.when((phase == 0) & (sub == slot) & sweep_done)
            def _():
                send_A.start()

            @pl.when((phase == 1) & (sub == slot) & sweep_done)
            def _():
                send_By.start()
                send_Bd.start()

            @pl.when(last_step)
            def _():
                send_A.wait_send()
                send_A.wait_recv()
                send_By.wait_send()
                send_By.wait_recv()
                send_Bd.wait_send()
                send_Bd.wait_recv()

        @pl.when(last_step)
        def _():
            rx.wait_send()
            ra.wait_send()
            f32 = jnp.float32
            out_ref[pl.ds(q_off, Q), :] = (
                pbufA[:, :].astype(f32) + rbufBy[:, :].astype(f32)
            ).astype(jnp.bfloat16)
            out_ref[pl.ds((1 - my_x) * Q, Q), :] = (
                rbufA[:, :].astype(f32) + rbufBd[:, :].astype(f32)
            ).astype(jnp.bfloat16)

    grid = (2, N_SB, E_LOCAL, N_FB)
    return pl.pallas_call(
        body,
        grid=grid,
        in_specs=[
            pl.BlockSpec(memory_space=pltpu.VMEM),
            pl.BlockSpec(memory_space=pltpu.VMEM),
            pl.BlockSpec((1, D, FB), lambda p, s, e, fb: (e, 0, fb)),
            pl.BlockSpec((1, FB, D), lambda p, s, e, fb: (e, fb, 0)),
        ],
        out_specs=pl.BlockSpec(memory_space=pltpu.VMEM),
        out_shape=jax.ShapeDtypeStruct((T, D), jnp.bfloat16),
        scratch_shapes=[
            pltpu.VMEM((Q, D), jnp.bfloat16),
            pltpu.VMEM((Q, 1), jnp.int32),
            pltpu.VMEM((Q, D), jnp.bfloat16),
            pltpu.VMEM((Q, D), jnp.bfloat16),
            pltpu.VMEM((Q, D), jnp.bfloat16),
            pltpu.VMEM((Q, D), jnp.bfloat16),
            pltpu.VMEM((Q, D), jnp.bfloat16),
            pltpu.SemaphoreType.DMA((2,)),
            pltpu.SemaphoreType.DMA((2,)),
            pltpu.SemaphoreType.DMA((N_SB,)),
            pltpu.SemaphoreType.DMA((N_SB,)),
            pltpu.SemaphoreType.DMA((N_SB,)),
            pltpu.SemaphoreType.DMA((N_SB,)),
            pltpu.SemaphoreType.DMA((N_SB,)),
            pltpu.SemaphoreType.DMA((N_SB,)),
        ],
        compiler_params=pltpu.CompilerParams(collective_id=0),
    )(x16, assign2d, W1, W2)


def kernel(x, assign, W1, W2):
    x16 = x.astype(jnp.bfloat16)
    assign2d = assign.reshape(T, 1)
    out16 = _fused(x16, assign2d, W1, W2)
    return out16.astype(jnp.float32)


# device time: 154124 ns/iter; 2.7863x vs baseline; 1.0481x over previous
import jax
import jax.numpy as jnp
from jax import lax
from jax.experimental import pallas as pl
from jax.experimental.pallas import tpu as pltpu

T = 2048
D = 1024
E_LOCAL = 4
F = 2048

Q = 1024
SB = 1024
N_SB = Q // SB
FB = 1024
N_FB = F // FB


def _fused(x16, assign2d, W1, W2):
    def body(x_ref, a_ref, w1_ref, w2_ref, out_ref,
             xp, ap, pbufA, pbufB, rbufA, rbufBy, rbufBd,
             xa_send, xa_recv, psA, prA, psBy, prBy, psBd, prBd):
        phase = pl.program_id(0)
        sub = pl.program_id(1)
        e = pl.program_id(2)
        fb = pl.program_id(3)
        my_x = lax.axis_index("x")
        my_y = lax.axis_index("y")
        xpeer = (1 - my_x, my_y)
        ypeer = (my_x, 1 - my_y)
        diag = (1 - my_x, 1 - my_y)

        q_off = my_x * Q

        rx = pltpu.make_async_remote_copy(
            src_ref=x_ref.at[pl.ds(q_off, Q), :],
            dst_ref=xp,
            send_sem=xa_send.at[0],
            recv_sem=xa_recv.at[0],
            device_id=ypeer,
            device_id_type=pl.DeviceIdType.MESH,
        )
        ra = pltpu.make_async_remote_copy(
            src_ref=a_ref.at[pl.ds(q_off, Q), :],
            dst_ref=ap,
            send_sem=xa_send.at[1],
            recv_sem=xa_recv.at[1],
            device_id=ypeer,
            device_id_type=pl.DeviceIdType.MESH,
        )

        @pl.when((phase == 0) & (sub == 0) & (e == 0) & (fb == 0))
        def _():
            barrier = pltpu.get_barrier_semaphore()
            for nbr in (xpeer, ypeer, diag):
                pl.semaphore_signal(barrier, inc=1, device_id=nbr,
                                    device_id_type=pl.DeviceIdType.MESH)
            pl.semaphore_wait(barrier, 3)
            rx.start()
            ra.start()

        @pl.when((phase == 1) & (sub == 0) & (e == 0) & (fb == 0))
        def _():
            rx.wait_recv()
            ra.wait_recv()

        row = q_off + sub * SB

        def tile(src_x, src_a, dst):
            xs = src_x
            h = jnp.maximum(
                jnp.dot(xs, w1_ref[0], preferred_element_type=jnp.float32),
                0.0)
            o = jnp.dot(h, w2_ref[0], preferred_element_type=jnp.float32)
            mask = (src_a == my_y * E_LOCAL + e)
            contrib = (mask.astype(jnp.float32) * o).astype(jnp.bfloat16)
            init = (e == 0) & (fb == 0)

            @pl.when(init)
            def _():
                dst[pl.ds(sub * SB, SB), :] = contrib

            @pl.when(jnp.logical_not(init))
            def _():
                dst[pl.ds(sub * SB, SB), :] += contrib

        @pl.when(phase == 0)
        def _():
            tile(x_ref[pl.ds(row, SB), :], a_ref[pl.ds(row, SB), :], pbufA)

        @pl.when(phase == 1)
        def _():
            tile(xp[pl.ds(sub * SB, SB), :], ap[pl.ds(sub * SB, SB), :], pbufB)

        sweep_done = (e == E_LOCAL - 1) & (fb == N_FB - 1)
        last_step = (phase == 1) & (sub == N_SB - 1) & sweep_done

        for slot in range(N_SB):
            sl = pl.ds(slot * SB, SB)
            send_A = pltpu.make_async_remote_copy(
                src_ref=pbufA.at[sl, :], dst_ref=rbufA.at[sl, :],
                send_sem=psA.at[slot], recv_sem=prA.at[slot],
                device_id=xpeer, device_id_type=pl.DeviceIdType.MESH,
            )
            send_By = pltpu.make_async_remote_copy(
                src_ref=pbufB.at[sl, :], dst_ref=rbufBy.at[sl, :],
                send_sem=psBy.at[slot], recv_sem=prBy.at[slot],
                device_id=ypeer, device_id_type=pl.DeviceIdType.MESH,
            )
            send_Bd = pltpu.make_async_remote_copy(
                src_ref=pbufB.at[sl, :], dst_ref=rbufBd.at[sl, :],
                send_sem=psBd.at[slot], recv_sem=prBd.at[slot],
                device_id=diag, device_id_type=pl.DeviceIdType.MESH,
            )

            @pl.when((phase == 0) & (sub == slot) & sweep_done)
            def _():
                send_A.start()

            @pl.when((phase == 1) & (sub == slot) & sweep_done)
            def _():
                send_By.start()
                send_Bd.start()

            @pl.when(last_step)
            def _():
                send_A.wait_send()
                send_A.wait_recv()
                send_By.wait_send()
                send_By.wait_recv()
                send_Bd.wait_send()
                send_Bd.wait_recv()

        @pl.when(last_step)
        def _():
            rx.wait_send()
            ra.wait_send()
            f32 = jnp.float32
            out_ref[pl.ds(q_off, Q), :] = (
                pbufA[:, :].astype(f32) + rbufBy[:, :].astype(f32)
            ).astype(jnp.bfloat16)
            out_ref[pl.ds((1 - my_x) * Q, Q), :] = (
                rbufA[:, :].astype(f32) + rbufBd[:, :].astype(f32)
            ).astype(jnp.bfloat16)

    grid = (2, N_SB, E_LOCAL, N_FB)
    return pl.pallas_call(
        body,
        grid=grid,
        in_specs=[
            pl.BlockSpec(memory_space=pltpu.VMEM),
            pl.BlockSpec(memory_space=pltpu.VMEM),
            pl.BlockSpec((1, D, FB), lambda p, s, e, fb: (e, 0, fb)),
            pl.BlockSpec((1, FB, D), lambda p, s, e, fb: (e, fb, 0)),
        ],
        out_specs=pl.BlockSpec(memory_space=pltpu.VMEM),
        out_shape=jax.ShapeDtypeStruct((T, D), jnp.bfloat16),
        scratch_shapes=[
            pltpu.VMEM((Q, D), jnp.bfloat16),
            pltpu.VMEM((Q, 1), jnp.int32),
            pltpu.VMEM((Q, D), jnp.bfloat16),
            pltpu.VMEM((Q, D), jnp.bfloat16),
            pltpu.VMEM((Q, D), jnp.bfloat16),
            pltpu.VMEM((Q, D), jnp.bfloat16),
            pltpu.VMEM((Q, D), jnp.bfloat16),
            pltpu.SemaphoreType.DMA((2,)),
            pltpu.SemaphoreType.DMA((2,)),
            pltpu.SemaphoreType.DMA((N_SB,)),
            pltpu.SemaphoreType.DMA((N_SB,)),
            pltpu.SemaphoreType.DMA((N_SB,)),
            pltpu.SemaphoreType.DMA((N_SB,)),
            pltpu.SemaphoreType.DMA((N_SB,)),
            pltpu.SemaphoreType.DMA((N_SB,)),
        ],
        compiler_params=pltpu.CompilerParams(
            collective_id=0, vmem_limit_bytes=100 * 1024 * 1024),
    )(x16, assign2d, W1, W2)


def kernel(x, assign, W1, W2):
    x16 = x.astype(jnp.bfloat16)
    assign2d = assign.reshape(T, 1)
    out16 = _fused(x16, assign2d, W1, W2)
    return out16.astype(jnp.float32)
